# baseline (speedup 1.0000x reference)
import jax
import jax.numpy as jnp
from jax import lax
from jax.experimental import pallas as pl
from jax.experimental.pallas import tpu as pltpu
from jax.experimental.pallas import tpu_sc as plsc

_B, _S, _D = 4, 8192, 1024
_NW = 32
_S_PER_W = _S // _NW
_CS = 16
_CHUNK = _CS * _D
_N_CHUNKS = _S_PER_W // _CS
_NXB = 6
_LOOK = 5
_NU = _N_CHUNKS * _B


def _sc_body(x_hbm, pos_hbm, out_hbm, *refs):
    pos_b = refs[0:2]
    xb = refs[2:2 + _NXB]
    psem = refs[2 + _NXB:4 + _NXB]
    lsem = refs[4 + _NXB:4 + 2 * _NXB]
    ssem = refs[4 + 2 * _NXB:4 + 3 * _NXB]

    wid = lax.axis_index("s") * 2 + lax.axis_index("c")
    s_base = wid * _S_PER_W

    def row0(ci):
        return s_base + ci * _CS

    def pos_load(ci):
        return pltpu.make_async_copy(
            pos_hbm.at[pl.ds(row0(ci), _CS), :], pos_b[ci % 2], psem[ci % 2])

    def x_load(u):
        ci, b = divmod(u, _B)
        return pltpu.make_async_copy(
            x_hbm.at[b, pl.ds(row0(ci), _CS), :], xb[u % _NXB], lsem[u % _NXB])

    def x_store(u):
        ci, b = divmod(u, _B)
        return pltpu.make_async_copy(
            xb[u % _NXB], out_hbm.at[b, pl.ds(row0(ci), _CS), :], ssem[u % _NXB])

    pos_load(0).start()
    for u in range(_LOOK):
        x_load(u).start()

    for u in range(_NU):
        ci, b = divmod(u, _B)
        k = u % _NXB
        if b == 0:
            pos_load(ci).wait()
            if ci + 1 < _N_CHUNKS:
                pos_load(ci + 1).start()
        if u + _LOOK < _NU:
            if u + _LOOK - _NXB >= 0:
                x_store(u + _LOOK - _NXB).wait()
            x_load(u + _LOOK).start()
        x_load(u).wait()

        buf = xb[k]
        pos = pos_b[ci % 2]

        @plsc.parallel_loop(0, _CHUNK, 16, unroll=8)
        def add_body(i):
            r = i // _D
            c = i % _D
            buf[r, pl.ds(c, 16)] = buf[r, pl.ds(c, 16)] + pos[r, pl.ds(c, 16)]

        x_store(u).start()

    for u in range(_NU - _NXB, _NU):
        x_store(u).wait()


def kernel(x, pos_table):
    mesh = plsc.VectorSubcoreMesh(core_axis_name="c", subcore_axis_name="s")
    scratch = (
        [pltpu.VMEM((_CS, _D), jnp.float32)] * 2
        + [pltpu.VMEM((_CS, _D), jnp.float32)] * _NXB
        + [pltpu.SemaphoreType.DMA] * (2 + 2 * _NXB)
    )
    k = pl.kernel(
        _sc_body,
        out_type=jax.ShapeDtypeStruct((_B, _S, _D), jnp.float32),
        mesh=mesh,
        scratch_types=scratch,
    )
    return k(x, pos_table)


# speedup vs baseline: 1.0808x; 1.0808x over previous
import jax
import jax.numpy as jnp
from jax import lax
from jax.experimental import pallas as pl
from jax.experimental.pallas import tpu as pltpu
from jax.experimental.pallas import tpu_sc as plsc

_B, _S, _D = 4, 8192, 1024
_NW = 32
_S_PER_W = _S // _NW
_CS = 16
_CHUNK = _CS * _D
_N_CHUNKS = _S_PER_W // _CS
_NXB = 6
_LOOK = 4
_NU = _N_CHUNKS * _B


def _sc_body(x_hbm, pos_hbm, out_hbm, *refs):
    pos_b = refs[0:2]
    xb = refs[2:2 + _NXB]
    psem = refs[2 + _NXB:4 + _NXB]
    lsem = refs[4 + _NXB:4 + 2 * _NXB]
    ssem = refs[4 + 2 * _NXB:4 + 3 * _NXB]

    wid = lax.axis_index("s") * 2 + lax.axis_index("c")
    s_base = wid * _S_PER_W

    def row0(ci):
        return s_base + ci * _CS

    def pos_load(ci):
        return pltpu.make_async_copy(
            pos_hbm.at[pl.ds(row0(ci), _CS), :], pos_b[ci % 2], psem[ci % 2])

    def x_load(u):
        ci, b = divmod(u, _B)
        return pltpu.make_async_copy(
            x_hbm.at[b, pl.ds(row0(ci), _CS), :], xb[u % _NXB], lsem[u % _NXB])

    def x_store_half(u, h):
        ci, b = divmod(u, _B)
        r0 = h * (_CS // 2)
        return pltpu.make_async_copy(
            xb[u % _NXB].at[pl.ds(r0, _CS // 2), :],
            out_hbm.at[b, pl.ds(row0(ci) + r0, _CS // 2), :], ssem[u % _NXB])

    def x_store_wait(u):
        x_store_half(u, 0).wait()
        x_store_half(u, 1).wait()

    pos_load(0).start()
    for u in range(_LOOK):
        x_load(u).start()

    for u in range(_NU):
        ci, b = divmod(u, _B)
        k = u % _NXB
        if b == 0:
            pos_load(ci).wait()
            if ci + 1 < _N_CHUNKS:
                pos_load(ci + 1).start()
        if u + _LOOK < _NU:
            if u + _LOOK - _NXB >= 0:
                x_store_wait(u + _LOOK - _NXB)
            x_load(u + _LOOK).start()
        x_load(u).wait()

        buf = xb[k]
        pos = pos_b[ci % 2]

        for h in range(2):
            r0 = h * (_CS // 2)

            @plsc.parallel_loop(r0 * _D, (r0 + _CS // 2) * _D, 16, unroll=8)
            def add_body(i):
                r = i // _D
                c = i % _D
                buf[r, pl.ds(c, 16)] = buf[r, pl.ds(c, 16)] + pos[r, pl.ds(c, 16)]

            x_store_half(u, h).start()

    for u in range(_NU - _NXB, _NU):
        x_store_wait(u)


def kernel(x, pos_table):
    mesh = plsc.VectorSubcoreMesh(core_axis_name="c", subcore_axis_name="s")
    scratch = (
        [pltpu.VMEM((_CS, _D), jnp.float32)] * 2
        + [pltpu.VMEM((_CS, _D), jnp.float32)] * _NXB
        + [pltpu.SemaphoreType.DMA] * (2 + 2 * _NXB)
    )
    k = pl.kernel(
        _sc_body,
        out_type=jax.ShapeDtypeStruct((_B, _S, _D), jnp.float32),
        mesh=mesh,
        scratch_types=scratch,
    )
    return k(x, pos_table)


# speedup vs baseline: 1.0883x; 1.0069x over previous
import jax
import jax.numpy as jnp
from jax import lax
from jax.experimental import pallas as pl
from jax.experimental.pallas import tpu as pltpu
from jax.experimental.pallas import tpu_sc as plsc

_B, _S, _D = 4, 8192, 1024
_NW = 32
_S_PER_W = _S // _NW
_CS = 16
_CHUNK = _CS * _D
_N_CHUNKS = _S_PER_W // _CS
_NXB = 6
_LOOK = 4
_NU = _N_CHUNKS * _B


def _sc_body(x_hbm, pos_hbm, out_hbm, *refs):
    pos_b = refs[0:2]
    xb = refs[2:2 + _NXB]
    psem = refs[2 + _NXB:4 + _NXB]
    lsem = refs[4 + _NXB:4 + 2 * _NXB]
    ssem = refs[4 + 2 * _NXB:4 + 3 * _NXB]

    wid = lax.axis_index("s") * 2 + lax.axis_index("c")
    s_base = wid * _S_PER_W

    def row0(ci):
        return s_base + ci * _CS

    def pos_load(ci):
        return pltpu.make_async_copy(
            pos_hbm.at[pl.ds(row0(ci), _CS), :], pos_b[ci % 2], psem[ci % 2])

    def x_load(u):
        ci, b = divmod(u, _B)
        return pltpu.make_async_copy(
            x_hbm.at[b, pl.ds(row0(ci), _CS), :], xb[u % _NXB], lsem[u % _NXB])

    def x_store(u):
        ci, b = divmod(u, _B)
        return pltpu.make_async_copy(
            xb[u % _NXB], out_hbm.at[b, pl.ds(row0(ci), _CS), :], ssem[u % _NXB])

    pos_load(0).start()
    for u in range(_LOOK):
        x_load(u).start()

    for u in range(_NU):
        ci, b = divmod(u, _B)
        k = u % _NXB
        if b == 0:
            pos_load(ci).wait()
            if ci + 1 < _N_CHUNKS:
                pos_load(ci + 1).start()
        if u + _LOOK < _NU:
            if u + _LOOK - _NXB >= 0:
                x_store(u + _LOOK - _NXB).wait()
            x_load(u + _LOOK).start()
        x_load(u).wait()

        buf = xb[k]
        pos = pos_b[ci % 2]

        @plsc.parallel_loop(0, _CHUNK, 16, unroll=16)
        def add_body(i):
            r = i // _D
            c = i % _D
            buf[r, pl.ds(c, 16)] = buf[r, pl.ds(c, 16)] + pos[r, pl.ds(c, 16)]

        x_store(u).start()

    for u in range(_NU - _NXB, _NU):
        x_store(u).wait()


def kernel(x, pos_table):
    mesh = plsc.VectorSubcoreMesh(core_axis_name="c", subcore_axis_name="s")
    scratch = (
        [pltpu.VMEM((_CS, _D), jnp.float32)] * 2
        + [pltpu.VMEM((_CS, _D), jnp.float32)] * _NXB
        + [pltpu.SemaphoreType.DMA] * (2 + 2 * _NXB)
    )
    k = pl.kernel(
        _sc_body,
        out_type=jax.ShapeDtypeStruct((_B, _S, _D), jnp.float32),
        mesh=mesh,
        scratch_types=scratch,
    )
    return k(x, pos_table)


# speedup vs baseline: 1.1177x; 1.0270x over previous
import jax
import jax.numpy as jnp
from jax import lax
from jax.experimental import pallas as pl
from jax.experimental.pallas import tpu as pltpu
from jax.experimental.pallas import tpu_sc as plsc

_B, _S, _D = 4, 8192, 1024
_NW = 32
_S_PER_W = _S // _NW
_CS = 16
_CHUNK = _CS * _D
_N_CHUNKS = _S_PER_W // _CS
_NXB = 6
_LOOK = 3
_NU = _N_CHUNKS * _B


def _sc_body(x_hbm, pos_hbm, out_hbm, *refs):
    pos_b = refs[0:2]
    xb = refs[2:2 + _NXB]
    psem = refs[2 + _NXB:4 + _NXB]
    lsem = refs[4 + _NXB:4 + 2 * _NXB]
    ssem = refs[4 + 2 * _NXB:4 + 3 * _NXB]

    wid = lax.axis_index("s") * 2 + lax.axis_index("c")
    s_base = wid * _S_PER_W

    def row0(ci):
        return s_base + ci * _CS

    def pos_load(ci):
        return pltpu.make_async_copy(
            pos_hbm.at[pl.ds(row0(ci), _CS), :], pos_b[ci % 2], psem[ci % 2])

    def x_load(u):
        ci, b = divmod(u, _B)
        return pltpu.make_async_copy(
            x_hbm.at[b, pl.ds(row0(ci), _CS), :], xb[u % _NXB], lsem[u % _NXB])

    def x_store(u):
        ci, b = divmod(u, _B)
        return pltpu.make_async_copy(
            xb[u % _NXB], out_hbm.at[b, pl.ds(row0(ci), _CS), :], ssem[u % _NXB])

    pos_load(0).start()
    for u in range(_LOOK):
        x_load(u).start()

    for u in range(_NU):
        ci, b = divmod(u, _B)
        k = u % _NXB
        if b == 0:
            pos_load(ci).wait()
            if ci + 1 < _N_CHUNKS:
                pos_load(ci + 1).start()
        if u + _LOOK < _NU:
            if u + _LOOK - _NXB >= 0:
                x_store(u + _LOOK - _NXB).wait()
            x_load(u + _LOOK).start()
        x_load(u).wait()

        buf = xb[k]
        pos = pos_b[ci % 2]

        @plsc.parallel_loop(0, _CHUNK, 16, unroll=8)
        def add_body(i):
            r = i // _D
            c = i % _D
            buf[r, pl.ds(c, 16)] = buf[r, pl.ds(c, 16)] + pos[r, pl.ds(c, 16)]

        x_store(u).start()

    for u in range(_NU - _NXB, _NU):
        x_store(u).wait()


def kernel(x, pos_table):
    mesh = plsc.VectorSubcoreMesh(core_axis_name="c", subcore_axis_name="s")
    scratch = (
        [pltpu.VMEM((_CS, _D), jnp.float32)] * 2
        + [pltpu.VMEM((_CS, _D), jnp.float32)] * _NXB
        + [pltpu.SemaphoreType.DMA] * (2 + 2 * _NXB)
    )
    k = pl.kernel(
        _sc_body,
        out_type=jax.ShapeDtypeStruct((_B, _S, _D), jnp.float32),
        mesh=mesh,
        scratch_types=scratch,
    )
    return k(x, pos_table)
